# baseline (speedup 1.0000x reference)
import jax
import jax.numpy as jnp
from jax import lax
from jax.experimental import pallas as pl
from jax.experimental.pallas import tpu as pltpu
from jax.experimental.pallas import tpu_sc as plsc

_CHUNK = 128
_LANES = 16
_K = 5
_NG = 2


def _word_embed_ln_sc(x3, table, gamma, beta, n_rows):
    num_w, nchunks, _ = x3.shape
    V, E = table.shape
    K = E // _LANES
    info = plsc.get_sparse_core_info()
    NC = info.num_cores
    rows_per_w = nchunks * _CHUNK
    nsuper = nchunks // _K
    group_rows = _K * _CHUNK

    def body(x_ref, table_ref, gamma_ref, beta_ref, out_ref,
             idx_v, grp0, grp1, gb_v, gsem0, gsem1, wsem0, wsem1):
        grps = (grp0, grp1)
        gsems = (gsem0, gsem1)
        wsems = (wsem0, wsem1)
        wid = lax.axis_index("s") * NC + lax.axis_index("c")
        base_row = wid * rows_per_w

        pltpu.sync_copy(x_ref.at[wid], idx_v)
        pltpu.sync_copy(gamma_ref, gb_v.at[pl.ds(0, E)])
        pltpu.sync_copy(beta_ref, gb_v.at[pl.ds(E, E)])
        gvs = [gb_v[pl.ds(k * _LANES, _LANES)] for k in range(K)]
        bvs = [gb_v[pl.ds(E + k * _LANES, _LANES)] for k in range(K)]

        def gather_group(s, grp, gsem):
            for j in range(_K):
                pltpu.async_copy(
                    table_ref.at[idx_v.at[s * _K + j]],
                    grp.at[pl.ds(j * _CHUNK, _CHUNK)], gsem)

        def drain_gathers(grp, gsem):
            for j in range(_K):
                pltpu.make_async_copy(
                    table_ref.at[idx_v.at[0]],
                    grp.at[pl.ds(j * _CHUNK, _CHUNK)], gsem).wait()

        def write_group(s, grp, wsem):
            for j in range(_K):
                dst = out_ref.at[
                    pl.ds(base_row + (s * _K + j) * _CHUNK, _CHUNK)]
                pltpu.async_copy(grp.at[pl.ds(j * _CHUNK, _CHUNK)], dst, wsem)

        def drain_writes(grp, wsem):
            dst = out_ref.at[pl.ds(base_row, group_rows)]
            pltpu.make_async_copy(grp, dst, wsem).wait()

        def compute_rows(buf, base):
            def g_body(g, carry):
                for l in range(_LANES):
                    r = base + g * _LANES + l
                    vs = [buf[r, pl.ds(k * _LANES, _LANES)]
                          for k in range(K)]
                    s = vs[0]
                    sq = vs[0] * vs[0]
                    for k in range(1, K):
                        s = s + vs[k]
                        sq = sq + vs[k] * vs[k]
                    total = jnp.sum(s)
                    ssq = jnp.sum(sq)
                    mean = total * (1.0 / E)
                    var = ssq * (1.0 / E) - mean * mean
                    var = jnp.maximum(var, 0.0) + 1e-12
                    i = lax.bitcast_convert_type(var, jnp.int32)
                    i = jnp.int32(0x5F3759DF) - lax.shift_right_logical(i, 1)
                    y = lax.bitcast_convert_type(i, jnp.float32)
                    xh = var * 0.5
                    for _ in range(3):
                        y = y * (1.5 - xh * y * y)
                    mb = mean * y
                    for k in range(K):
                        t = vs[k] * y - mb
                        buf[r, pl.ds(k * _LANES, _LANES)] = (
                            t * gvs[k] + bvs[k])
                return carry

            lax.fori_loop(0, _CHUNK // _LANES, g_body, 0)

        def super_step(s, g):
            grp, gsem, wsem = grps[g], gsems[g], wsems[g]
            drain_gathers(grp, gsem)
            for j in range(_K):
                compute_rows(grp, j * _CHUNK)
            write_group(s, grp, wsem)

            @pl.when(s + _NG < nsuper)
            def _():
                drain_writes(grp, wsem)
                gather_group(s + _NG, grp, gsem)

        for g in range(_NG):
            gather_group(g, grps[g], gsems[g])

        def loop_body(i, carry):
            for u in range(_NG):
                super_step(i * _NG + u, u)
            return carry

        lax.fori_loop(0, nsuper // _NG, loop_body, 0)
        for g in range(_NG):
            drain_writes(grps[g], wsems[g])

    mesh = plsc.VectorSubcoreMesh(core_axis_name="c", subcore_axis_name="s")
    kern = pl.kernel(
        body,
        mesh=mesh,
        compiler_params=pltpu.CompilerParams(
            needs_layout_passes=False, use_tc_tiling_on_sc=False),
        out_type=jax.ShapeDtypeStruct((n_rows, E), jnp.float32),
        scratch_types=(
            [pltpu.VMEM((nchunks, _CHUNK), jnp.int32)]
            + [pltpu.VMEM((_K * _CHUNK, E), jnp.float32)
               for _ in range(_NG)]
            + [pltpu.VMEM((2 * E,), jnp.float32)]
            + [pltpu.SemaphoreType.DMA for _ in range(2 * _NG)]
        ),
    )
    return kern(x3, table, gamma, beta)


def kernel(x, table, gamma, beta):
    B, L = x.shape
    V, E = table.shape
    N = B * L
    info = plsc.get_sparse_core_info()
    num_w = info.num_cores * info.num_subcores
    rows_per_w = N // num_w
    nchunks = rows_per_w // _CHUNK
    x3 = x.reshape(num_w, nchunks, _CHUNK)
    out = _word_embed_ln_sc(x3, table, gamma, beta, N)
    return out.reshape(B, L, E)


# speedup vs baseline: 1.0086x; 1.0086x over previous
import jax
import jax.numpy as jnp
from jax import lax
from jax.experimental import pallas as pl
from jax.experimental.pallas import tpu as pltpu
from jax.experimental.pallas import tpu_sc as plsc

_CHUNK = 640
_LANES = 16
_NB = 2


def _word_embed_ln_sc(x3, table, gamma, beta, n_rows):
    num_w, nchunks, _ = x3.shape
    V, E = table.shape
    K = E // _LANES
    info = plsc.get_sparse_core_info()
    NC = info.num_cores
    rows_per_w = nchunks * _CHUNK

    def body(x_ref, table_ref, gamma_ref, beta_ref, out_ref,
             idx_v, buf0, buf1, gb_v, gsem0, gsem1, wsem0, wsem1):
        bufs = (buf0, buf1)
        gsems = (gsem0, gsem1)
        wsems = (wsem0, wsem1)
        wid = lax.axis_index("s") * NC + lax.axis_index("c")
        base_row = wid * rows_per_w

        pltpu.sync_copy(x_ref.at[wid], idx_v)
        pltpu.sync_copy(gamma_ref, gb_v.at[pl.ds(0, E)])
        pltpu.sync_copy(beta_ref, gb_v.at[pl.ds(E, E)])
        gvs = [gb_v[pl.ds(k * _LANES, _LANES)] for k in range(K)]
        bvs = [gb_v[pl.ds(E + k * _LANES, _LANES)] for k in range(K)]

        def gather_start(c, buf, sem):
            pltpu.async_copy(table_ref.at[idx_v.at[c]], buf, sem)

        def gather_wait(buf, sem):
            pltpu.make_async_copy(table_ref.at[idx_v.at[0]], buf, sem).wait()

        def write_start(c, buf, sem):
            dst = out_ref.at[pl.ds(base_row + c * _CHUNK, _CHUNK)]
            pltpu.async_copy(buf, dst, sem)

        def write_wait(buf, sem):
            dst = out_ref.at[pl.ds(base_row, _CHUNK)]
            pltpu.make_async_copy(buf, dst, sem).wait()

        def compute(buf):
            def g_body(g, carry):
                for l in range(_LANES):
                    r = g * _LANES + l
                    vs = [buf[r, pl.ds(k * _LANES, _LANES)]
                          for k in range(K)]
                    s = vs[0]
                    sq = vs[0] * vs[0]
                    for k in range(1, K):
                        s = s + vs[k]
                        sq = sq + vs[k] * vs[k]
                    total = jnp.sum(s)
                    ssq = jnp.sum(sq)
                    mean = total * (1.0 / E)
                    var = ssq * (1.0 / E) - mean * mean
                    var = jnp.maximum(var, 0.0) + 1e-12
                    i = lax.bitcast_convert_type(var, jnp.int32)
                    i = jnp.int32(0x5F3759DF) - lax.shift_right_logical(i, 1)
                    y = lax.bitcast_convert_type(i, jnp.float32)
                    xh = var * 0.5
                    for _ in range(3):
                        y = y * (1.5 - xh * y * y)
                    mb = mean * y
                    for k in range(K):
                        t = vs[k] * y - mb
                        buf[r, pl.ds(k * _LANES, _LANES)] = (
                            t * gvs[k] + bvs[k])
                return carry

            lax.fori_loop(0, _CHUNK // _LANES, g_body, 0)

        def step(c, b):
            buf, gsem, wsem = bufs[b], gsems[b], wsems[b]
            gather_wait(buf, gsem)
            compute(buf)
            write_start(c, buf, wsem)

            @pl.when(c + _NB < nchunks)
            def _():
                write_wait(buf, wsem)
                gather_start(c + _NB, buf, gsem)

        for b in range(_NB):
            gather_start(b, bufs[b], gsems[b])

        def loop_body(i, carry):
            for b in range(_NB):
                step(i * _NB + b, b)
            return carry

        lax.fori_loop(0, nchunks // _NB, loop_body, 0)
        for b in range(_NB):
            write_wait(bufs[b], wsems[b])

    mesh = plsc.VectorSubcoreMesh(core_axis_name="c", subcore_axis_name="s")
    kern = pl.kernel(
        body,
        mesh=mesh,
        compiler_params=pltpu.CompilerParams(
            needs_layout_passes=False, use_tc_tiling_on_sc=False),
        out_type=jax.ShapeDtypeStruct((n_rows, E), jnp.float32),
        scratch_types=(
            [pltpu.VMEM((nchunks, _CHUNK), jnp.int32)]
            + [pltpu.VMEM((_CHUNK, E), jnp.float32)
               for _ in range(_NB)]
            + [pltpu.VMEM((2 * E,), jnp.float32)]
            + [pltpu.SemaphoreType.DMA for _ in range(2 * _NB)]
        ),
    )
    return kern(x3, table, gamma, beta)


def kernel(x, table, gamma, beta):
    B, L = x.shape
    V, E = table.shape
    N = B * L
    info = plsc.get_sparse_core_info()
    num_w = info.num_cores * info.num_subcores
    rows_per_w = N // num_w
    nchunks = rows_per_w // _CHUNK
    x3 = x.reshape(num_w, nchunks, _CHUNK)
    out = _word_embed_ln_sc(x3, table, gamma, beta, N)
    return out.reshape(B, L, E)


# speedup vs baseline: 1.0551x; 1.0462x over previous
import jax
import jax.numpy as jnp
from jax import lax
from jax.experimental import pallas as pl
from jax.experimental.pallas import tpu as pltpu
from jax.experimental.pallas import tpu_sc as plsc

_CHUNK = 128
_LANES = 16
_DEPTH = 2


def _word_embed_ln_sc(x3, table, gamma, beta, n_rows):
    num_w, nchunks, _ = x3.shape
    V, E = table.shape
    K = E // _LANES
    info = plsc.get_sparse_core_info()
    NC = info.num_cores
    rows_per_w = nchunks * _CHUNK

    def body(x_ref, table_ref, gamma_ref, beta_ref, out_ref,
             idx_v, *rest):
        ins = rest[:_DEPTH]
        outs = rest[_DEPTH:_DEPTH + 2]
        gb_v = rest[_DEPTH + 2]
        gsems = rest[_DEPTH + 3:2 * _DEPTH + 3]
        osems = rest[2 * _DEPTH + 3:2 * _DEPTH + 5]
        wid = lax.axis_index("s") * NC + lax.axis_index("c")
        base_row = wid * rows_per_w

        pltpu.sync_copy(x_ref.at[wid], idx_v)
        pltpu.sync_copy(gamma_ref, gb_v.at[pl.ds(0, E)])
        pltpu.sync_copy(beta_ref, gb_v.at[pl.ds(E, E)])
        gvs = [gb_v[pl.ds(k * _LANES, _LANES)] for k in range(K)]
        bvs = [gb_v[pl.ds(E + k * _LANES, _LANES)] for k in range(K)]

        def gather_start(c, buf, sem):
            pltpu.async_copy(table_ref.at[idx_v.at[c]], buf, sem)

        def gather_wait(c, buf, sem):
            pltpu.make_async_copy(table_ref.at[idx_v.at[c]], buf, sem).wait()

        def out_start(c, buf, sem):
            dst = out_ref.at[pl.ds(base_row + c * _CHUNK, _CHUNK)]
            pltpu.async_copy(buf, dst, sem)

        def out_wait(buf, sem):
            dst = out_ref.at[pl.ds(base_row, _CHUNK)]
            pltpu.make_async_copy(buf, dst, sem).wait()

        def compute(in_buf, out_buf):
            def g_body(g, carry):
                for l in range(_LANES):
                    r = g * _LANES + l
                    vs = [in_buf[r, pl.ds(k * _LANES, _LANES)]
                          for k in range(K)]
                    s = vs[0]
                    sq = vs[0] * vs[0]
                    for k in range(1, K):
                        s = s + vs[k]
                        sq = sq + vs[k] * vs[k]
                    total = jnp.sum(s)
                    ssq = jnp.sum(sq)
                    mean = total * (1.0 / E)
                    var = ssq * (1.0 / E) - mean * mean
                    var = jnp.maximum(var, 0.0) + 1e-12
                    i = lax.bitcast_convert_type(var, jnp.int32)
                    i = jnp.int32(0x5F3759DF) - lax.shift_right_logical(i, 1)
                    y = lax.bitcast_convert_type(i, jnp.float32)
                    xh = var * 0.5
                    for _ in range(3):
                        y = y * (1.5 - xh * y * y)
                    mb = mean * y
                    for k in range(K):
                        t = vs[k] * y - mb
                        out_buf[r, pl.ds(k * _LANES, _LANES)] = (
                            t * gvs[k] + bvs[k])
                return carry

            lax.fori_loop(0, _CHUNK // _LANES, g_body, 0)

        def step(c, inb, outb, gsem, osem):
            gather_wait(c, inb, gsem)

            @pl.when(c >= 2)
            def _():
                out_wait(outb, osem)

            compute(inb, outb)
            out_start(c, outb, osem)

            @pl.when(c + _DEPTH < nchunks)
            def _():
                gather_start(c + _DEPTH, inb, gsem)

        for d in range(_DEPTH):
            gather_start(d, ins[d], gsems[d])

        def loop_body(i, carry):
            for d in range(_DEPTH):
                c = i * _DEPTH + d
                step(c, ins[d], outs[d % 2], gsems[d], osems[d % 2])
            return carry

        lax.fori_loop(0, nchunks // _DEPTH, loop_body, 0)
        out_wait(outs[0], osems[0])
        out_wait(outs[1], osems[1])

    mesh = plsc.VectorSubcoreMesh(core_axis_name="c", subcore_axis_name="s")
    kern = pl.kernel(
        body,
        mesh=mesh,
        compiler_params=pltpu.CompilerParams(
            needs_layout_passes=False, use_tc_tiling_on_sc=False),
        out_type=jax.ShapeDtypeStruct((n_rows, E), jnp.float32),
        scratch_types=(
            [pltpu.VMEM((nchunks, _CHUNK), jnp.int32)]
            + [pltpu.VMEM((_CHUNK, E), jnp.float32)
               for _ in range(_DEPTH)]
            + [pltpu.VMEM((_CHUNK, E), jnp.float32)
               for _ in range(2)]
            + [pltpu.VMEM((2 * E,), jnp.float32)]
            + [pltpu.SemaphoreType.DMA for _ in range(_DEPTH + 2)]
        ),
    )
    return kern(x3, table, gamma, beta)


def kernel(x, table, gamma, beta):
    B, L = x.shape
    V, E = table.shape
    N = B * L
    info = plsc.get_sparse_core_info()
    num_w = info.num_cores * info.num_subcores
    rows_per_w = N // num_w
    nchunks = rows_per_w // _CHUNK
    x3 = x.reshape(num_w, nchunks, _CHUNK)
    out = _word_embed_ln_sc(x3, table, gamma, beta, N)
    return out.reshape(B, L, E)
